# Initial kernel scaffold; baseline (speedup 1.0000x reference)
import jax
import jax.numpy as jnp
from jax.experimental import pallas as pl


def kernel(x, qkv_w, qkv_dw_w, proj_w, temperature, attn_mix, ln_w, ln_b, ff_in_w, g1_pw, g1_dw, g2_pw, g2_dw, g3_pw, g3_dw):
    raise NotImplementedError("write your pallas kernel here")


# speedup vs baseline: 2.1649x; 2.1649x over previous
import functools

import jax
import jax.numpy as jnp
from jax.experimental import pallas as pl

HEADS = 4
F32 = jnp.float32


def _shift(x, off):
    if off == 0:
        return x
    C, L = x.shape
    z = jnp.zeros((C, abs(off)), x.dtype)
    if off > 0:
        return jnp.concatenate([x[:, off:], z], axis=1)
    return jnp.concatenate([z, x[:, :off]], axis=1)


def _row_col(base, rows, W):
    L = rows * W
    ii = jax.lax.broadcasted_iota(jnp.int32, (8, L), 1)[:1]
    return base + ii // W, ii % W


def _dw3x3(x, w9_ref, row, col, W, H):
    acc = None
    t = 0
    for dr in (-1, 0, 1):
        rm = ((row + dr) >= 0) & ((row + dr) < H)
        for dc in (-1, 0, 1):
            cm = ((col + dc) >= 0) & ((col + dc) < W)
            m = (rm & cm).astype(x.dtype)
            tap = _shift(x, dr * W + dc) * m
            contrib = tap * w9_ref[t, :, :]
            acc = contrib if acc is None else acc + contrib
            t += 1
    return acc


def _dw3x3_half(pb, w9_ref, row, col, W, H):
    rpar = row % 2
    cpar = col % 2
    hrow = row // 2
    hcol = col // 2
    Hh, Wh = H // 2, W // 2
    ceven = (cpar == 0)
    reven = (rpar == 0)
    acc = None
    for dj in (-1, 0, 1):
        cm = ((hcol + dj) >= 0) & ((hcol + dj) < Wh)
        cdj = jnp.where(ceven, _shift(pb, 2 * dj), _shift(pb, 2 * dj - 1))
        for di in (-1, 0, 1):
            rm = ((hrow + di) >= 0) & ((hrow + di) < Hh)
            m = (rm & cm).astype(pb.dtype)
            tap = jnp.where(reven, _shift(cdj, 2 * di * W),
                            _shift(cdj, (2 * di - 1) * W))
            t = (di + 1) * 3 + (dj + 1)
            contrib = tap * m * w9_ref[t, :, :]
            acc = contrib if acc is None else acc + contrib
    return acc


def _k1_body(xu, xm, xd, wqkv, wdw9, o_v, o_s, o_nq, o_nk, *, T, W, H, C):
    i = pl.program_id(0)
    XE = jnp.concatenate([xu[...], xm[...], xd[...]], axis=1)
    base = i * T - 1
    row, col = _row_col(base, T + 2, W)
    qkv1 = jnp.dot(wqkv[...], XE, preferred_element_type=F32)
    qkv = _dw3x3(qkv1, wdw9, row, col, W, H)
    qkv = qkv[:, W:(T + 1) * W]
    q = qkv[:C]
    k = qkv[C:2 * C]
    v = qkv[2 * C:]
    o_v[...] = v

    @pl.when(i == 0)
    def _init():
        o_s[...] = jnp.zeros_like(o_s)
        o_nq[...] = jnp.zeros_like(o_nq)
        o_nk[...] = jnp.zeros_like(o_nk)

    o_s[...] += jax.lax.dot_general(q, k, (((1,), (1,)), ((), ())),
                                    preferred_element_type=F32)
    o_nq[...] += jnp.broadcast_to(jnp.sum(q * q, axis=1, keepdims=True),
                                  o_nq.shape)
    o_nk[...] += jnp.broadcast_to(jnp.sum(k * k, axis=1, keepdims=True),
                                  o_nk.shape)


def _k2_body(s_ref, nq_ref, nk_ref, temp_ref, mix_ref, o_ref, *, C):
    ch = C // HEADS
    CC = ch * ch
    ks = (CC * 1 // 2, CC * 2 // 3, CC * 3 // 4, CC * 4 // 5)
    invq = 1.0 / jnp.maximum(jnp.sqrt(nq_ref[:, :1]), 1e-12)
    invk = 1.0 / jnp.maximum(jnp.sqrt(nk_ref[:, :1]), 1e-12)
    i0 = jax.lax.broadcasted_iota(jnp.int32, (ch, ch), 0)
    i1 = jax.lax.broadcasted_iota(jnp.int32, (ch, ch), 1)
    li = i0 * ch + i1
    strips = []
    for h in range(HEADS):
        lo, hi = h * ch, (h + 1) * ch
        sblk = s_ref[lo:hi, lo:hi]
        dk = jnp.where(i0 == i1, jnp.broadcast_to(invk[lo:hi], (ch, ch)), 0.0)
        A = (invq[lo:hi] * sblk)
        A = jnp.dot(A, dk, preferred_element_type=F32) * temp_ref[h, 0]
        gt = A[None, None, :, :] > A[:, :, None, None]
        tie = (A[None, None, :, :] == A[:, :, None, None]) & \
              (li[None, None, :, :] < li[:, :, None, None])
        rank = jnp.sum((gt | tie).astype(F32), axis=(2, 3))
        e = jnp.exp(A - jnp.max(A))
        spa = jnp.zeros((ch, ch), F32)
        for j, kk in enumerate(ks):
            em = e * (rank < kk).astype(F32)
            spa = spa + mix_ref[j, 0] * em / jnp.sum(em)
        pieces = []
        if lo > 0:
            pieces.append(jnp.zeros((ch, lo), F32))
        pieces.append(spa)
        if C - hi > 0:
            pieces.append(jnp.zeros((ch, C - hi), F32))
        strips.append(jnp.concatenate(pieces, axis=1) if len(pieces) > 1
                      else pieces[0])
    o_ref[...] = jnp.concatenate(strips, axis=0)


def _k3_body(xu, xm, xd, vu, vm, vd, spa, projw, lnw, lnb, ffw,
             g1p, g1d9, g2p, g2d9, g3p, g3d9, o_ref, *, T, W, H, C):
    i = pl.program_id(0)
    XE = jnp.concatenate([xu[...], xm[...], xd[...]], axis=1)
    VE = jnp.concatenate([vu[...], vm[...], vd[...]], axis=1)
    base = i * T - 4
    row, col = _row_col(base, T + 8, W)

    o = jnp.dot(spa[...], VE, preferred_element_type=F32)
    y = jnp.dot(projw[...], o, preferred_element_type=F32) + XE

    mu = jnp.mean(y, axis=0, keepdims=True)
    var = jnp.mean((y - mu) ** 2, axis=0, keepdims=True)
    yn = (y - mu) * jax.lax.rsqrt(var + 1e-6)
    yn = lnw[...] * yn + lnb[...]

    t = jnp.dot(ffw[...], yn, preferred_element_type=F32)
    t = 0.5 * t * (1.0 + jax.lax.erf(t * 0.7071067811865476))
    x1 = t[:C]
    x2 = t[C:]

    a1 = jnp.dot(g1p[...], x1, preferred_element_type=F32)
    x1d = _dw3x3(a1, g1d9, row, col, W, H)

    pcol = jnp.where(col % 2 == 0, _shift(x2, 1), _shift(x2, -1))
    s2 = x2 + pcol
    prow = jnp.where(row % 2 == 0, _shift(s2, W), _shift(s2, -W))
    pb = (s2 + prow) * 0.25
    a2 = jnp.dot(g2p[...], pb, preferred_element_type=F32)
    x2u = _dw3x3_half(a2, g2d9, row, col, W, H)

    a3 = jnp.dot(g3p[...], x1d * x2u, preferred_element_type=F32)
    ff = _dw3x3(a3, g3d9, row, col, W, H)
    out = ff + y
    o_ref[...] = out[:, 4 * W:(T + 4) * W]


def _dw9(w):
    C = w.shape[0]
    return jnp.transpose(w.reshape(C, 9), (1, 0))[:, :, None]


def kernel(x, qkv_w, qkv_dw_w, proj_w, temperature, attn_mix, ln_w, ln_b,
           ff_in_w, g1_pw, g1_dw, g2_pw, g2_dw, g3_pw, g3_dw):
    b, C, H, W = x.shape
    N = H * W
    xf = x.reshape(C, N)
    wqkv = qkv_w.reshape(3 * C, C)
    wdw9 = _dw9(qkv_dw_w)
    projw = proj_w.reshape(C, C)
    ffw = ff_in_w.reshape(2 * C, C)
    g1p = g1_pw.reshape(C, C)
    g2p = g2_pw.reshape(C, C)
    g3p = g3_pw.reshape(C, C)
    g1d9 = _dw9(g1_dw)
    g2d9 = _dw9(g2_dw)
    g3d9 = _dw9(g3_dw)
    lnw = ln_w.reshape(C, 1)
    lnb = ln_b.reshape(C, 1)
    temp = temperature.reshape(HEADS, 1)
    mix = attn_mix.reshape(4, 1)

    T1 = 8

    def up_map(i):
        return (0, jnp.maximum(i * T1 - 1, 0))

    def dn_map(i):
        return (0, jnp.minimum(i * T1 + T1, H - 1))

    halo_spec = pl.BlockSpec((C, W), up_map)
    halo_spec_dn = pl.BlockSpec((C, W), dn_map)
    main_spec = pl.BlockSpec((C, T1 * W), lambda i: (0, i))
    full2 = lambda s: pl.BlockSpec(s, lambda i: (0, 0))
    full3 = lambda s: pl.BlockSpec(s, lambda i: (0, 0, 0))

    v, S, nq, nk = pl.pallas_call(
        functools.partial(_k1_body, T=T1, W=W, H=H, C=C),
        grid=(H // T1,),
        in_specs=[halo_spec, main_spec, halo_spec_dn,
                  full2((3 * C, C)), full3((9, 3 * C, 1))],
        out_specs=[main_spec, full2((C, C)), full2((C, 128)),
                   full2((C, 128))],
        out_shape=[jax.ShapeDtypeStruct((C, N), F32),
                   jax.ShapeDtypeStruct((C, C), F32),
                   jax.ShapeDtypeStruct((C, 128), F32),
                   jax.ShapeDtypeStruct((C, 128), F32)],
    )(xf, xf, xf, wqkv, wdw9)

    spa = pl.pallas_call(
        functools.partial(_k2_body, C=C),
        grid=(1,),
        in_specs=[full2((C, C)), full2((C, 128)), full2((C, 128)),
                  full2((HEADS, 1)), full2((4, 1))],
        out_specs=full2((C, C)),
        out_shape=jax.ShapeDtypeStruct((C, C), F32),
    )(S, nq, nk, temp, mix)

    T3 = 8
    nb3 = T3 // 4
    nrb4 = H // 4

    def up3(i):
        return (0, jnp.maximum(i * nb3 - 1, 0))

    def dn3(i):
        return (0, jnp.minimum(i * nb3 + nb3, nrb4 - 1))

    halo3u = pl.BlockSpec((C, 4 * W), up3)
    halo3d = pl.BlockSpec((C, 4 * W), dn3)
    main3 = pl.BlockSpec((C, T3 * W), lambda i: (0, i))

    out = pl.pallas_call(
        functools.partial(_k3_body, T=T3, W=W, H=H, C=C),
        grid=(H // T3,),
        in_specs=[halo3u, main3, halo3d,
                  halo3u, main3, halo3d,
                  full2((C, C)), full2((C, C)), full2((C, 1)), full2((C, 1)),
                  full2((2 * C, C)),
                  full2((C, C)), full3((9, C, 1)),
                  full2((C, C)), full3((9, C, 1)),
                  full2((C, C)), full3((9, C, 1))],
        out_specs=main3,
        out_shape=jax.ShapeDtypeStruct((C, N), F32),
    )(xf, xf, xf, v, v, v, spa, projw, lnw, lnb, ffw,
      g1p, g1d9, g2p, g2d9, g3p, g3d9)

    return out.reshape(b, C, H, W)


# speedup vs baseline: 2.6065x; 1.2040x over previous
import functools

import jax
import jax.numpy as jnp
from jax.experimental import pallas as pl

HEADS = 4
F32 = jnp.float32


def _shift(x, off):
    if off == 0:
        return x
    C, L = x.shape
    z = jnp.zeros((C, abs(off)), x.dtype)
    if off > 0:
        return jnp.concatenate([x[:, off:], z], axis=1)
    return jnp.concatenate([z, x[:, :off]], axis=1)


def _row_col(base, rows, W):
    L = rows * W
    ii = jax.lax.broadcasted_iota(jnp.int32, (8, L), 1)[:1]
    return base + ii // W, ii % W


def _dw3x3(x, w9_ref, cl, cr, W):
    xl = _shift(x, -1) * cl
    xr = _shift(x, 1) * cr
    acc = None
    for dr in (-1, 0, 1):
        for xs, dc in ((xl, 0), (x, 1), (xr, 2)):
            tap = _shift(xs, dr * W)
            contrib = tap * w9_ref[(dr + 1) * 3 + dc, :, :]
            acc = contrib if acc is None else acc + contrib
    return acc


def _dw3x3_half(pb, w9_ref, row, col, W, H):
    hcol = col // 2
    Wh = W // 2
    ceven = (col % 2) == 0
    reven = (row % 2) == 0
    clh = (hcol >= 1).astype(pb.dtype)
    crh = (hcol <= Wh - 2).astype(pb.dtype)
    cvar = (
        jnp.where(ceven, _shift(pb, -2), _shift(pb, -3)) * clh,
        jnp.where(ceven, pb, _shift(pb, -1)),
        jnp.where(ceven, _shift(pb, 2), _shift(pb, 1)) * crh,
    )
    acc = None
    for di in (-1, 0, 1):
        for dj in (0, 1, 2):
            cdj = cvar[dj]
            tap = jnp.where(reven, _shift(cdj, 2 * di * W),
                            _shift(cdj, (2 * di - 1) * W))
            contrib = tap * w9_ref[(di + 1) * 3 + dj, :, :]
            acc = contrib if acc is None else acc + contrib
    return acc


def _k1_body(xu, xm, xd, wqkv, wdw9, o_v, o_s, o_nq, o_nk, *, T, W, H, C):
    i = pl.program_id(0)
    XE = jnp.concatenate([xu[...], xm[...], xd[...]], axis=1)
    base = i * T - 1
    row, col = _row_col(base, T + 2, W)
    rv = ((row >= 0) & (row < H)).astype(F32)
    cl = (col >= 1).astype(F32)
    cr = (col <= W - 2).astype(F32)
    qkv1 = jnp.dot(wqkv[...], XE.astype(jnp.bfloat16),
                   preferred_element_type=F32) * rv
    lo_m, hi_m = W, (T + 1) * W
    q = _dw3x3(qkv1[:C], wdw9.at[:, :C], cl, cr, W)[:, lo_m:hi_m]
    k = _dw3x3(qkv1[C:2 * C], wdw9.at[:, C:2 * C], cl, cr, W)[:, lo_m:hi_m]
    v = _dw3x3(qkv1[2 * C:], wdw9.at[:, 2 * C:], cl, cr, W)[:, lo_m:hi_m]
    o_v[...] = v

    @pl.when(i == 0)
    def _init():
        o_s[...] = jnp.zeros_like(o_s)
        o_nq[...] = jnp.zeros_like(o_nq)
        o_nk[...] = jnp.zeros_like(o_nk)

    o_s[...] += jax.lax.dot_general(q.astype(jnp.bfloat16),
                                    k.astype(jnp.bfloat16),
                                    (((1,), (1,)), ((), ())),
                                    preferred_element_type=F32)
    o_nq[...] += jnp.broadcast_to(jnp.sum(q * q, axis=1, keepdims=True),
                                  o_nq.shape)
    o_nk[...] += jnp.broadcast_to(jnp.sum(k * k, axis=1, keepdims=True),
                                  o_nk.shape)


def _k2_body(s_ref, nq_ref, nk_ref, temp_ref, mix_ref, o_ref, *, C):
    ch = C // HEADS
    CC = ch * ch
    ks = (CC * 1 // 2, CC * 2 // 3, CC * 3 // 4, CC * 4 // 5)
    invq = 1.0 / jnp.maximum(jnp.sqrt(nq_ref[:, :1]), 1e-12)
    invk = 1.0 / jnp.maximum(jnp.sqrt(nk_ref[:, :1]), 1e-12)
    i0 = jax.lax.broadcasted_iota(jnp.int32, (ch, ch), 0)
    i1 = jax.lax.broadcasted_iota(jnp.int32, (ch, ch), 1)
    li = i0 * ch + i1
    strips = []
    for h in range(HEADS):
        lo, hi = h * ch, (h + 1) * ch
        sblk = s_ref[lo:hi, lo:hi]
        dk = jnp.where(i0 == i1, jnp.broadcast_to(invk[lo:hi], (ch, ch)), 0.0)
        A = (invq[lo:hi] * sblk)
        A = jnp.dot(A, dk, preferred_element_type=F32) * temp_ref[h, 0]
        gt = A[None, None, :, :] > A[:, :, None, None]
        tie = (A[None, None, :, :] == A[:, :, None, None]) & \
              (li[None, None, :, :] < li[:, :, None, None])
        rank = jnp.sum((gt | tie).astype(F32), axis=(2, 3))
        e = jnp.exp(A - jnp.max(A))
        spa = jnp.zeros((ch, ch), F32)
        for j, kk in enumerate(ks):
            em = e * (rank < kk).astype(F32)
            spa = spa + mix_ref[j, 0] * em / jnp.sum(em)
        pieces = []
        if lo > 0:
            pieces.append(jnp.zeros((ch, lo), F32))
        pieces.append(spa)
        if C - hi > 0:
            pieces.append(jnp.zeros((ch, C - hi), F32))
        strips.append(jnp.concatenate(pieces, axis=1) if len(pieces) > 1
                      else pieces[0])
    o_ref[...] = jnp.concatenate(strips, axis=0)


def _k3_body(xu, xm, xd, vu, vm, vd, spa, projw, lnw, lnb, ffw,
             g1p, g1d9, g2p, g2d9, g3p, g3d9, o_ref, *, T, W, H, C):
    i = pl.program_id(0)
    XE = jnp.concatenate([xu[...], xm[...], xd[...]], axis=1)
    VE = jnp.concatenate([vu[...], vm[...], vd[...]], axis=1)
    base = i * T - 4
    row, col = _row_col(base, T + 8, W)
    rv = ((row >= 0) & (row < H)).astype(F32)
    cl = (col >= 1).astype(F32)
    cr = (col <= W - 2).astype(F32)
    bf = jnp.bfloat16

    o = jnp.dot(spa[...].astype(bf), VE.astype(bf), preferred_element_type=F32)
    y = jnp.dot(projw[...], o.astype(bf), preferred_element_type=F32) + XE

    mu = jnp.mean(y, axis=0, keepdims=True)
    var = jnp.mean((y - mu) ** 2, axis=0, keepdims=True)
    yn = (y - mu) * jax.lax.rsqrt(var + 1e-6)
    yn = lnw[...] * yn + lnb[...]

    t = jnp.dot(ffw[...], yn.astype(bf), preferred_element_type=F32)
    t = 0.5 * t * (1.0 + jax.lax.erf(t * 0.7071067811865476))
    x1 = t[:C]
    x2 = t[C:]

    a1 = jnp.dot(g1p[...], x1.astype(bf), preferred_element_type=F32) * rv
    x1d = _dw3x3(a1, g1d9, cl, cr, W)

    pcol = jnp.where(col % 2 == 0, _shift(x2, 1), _shift(x2, -1))
    s2 = x2 + pcol
    prow = jnp.where(row % 2 == 0, _shift(s2, W), _shift(s2, -W))
    pb = (s2 + prow) * 0.25
    a2 = jnp.dot(g2p[...], pb.astype(bf), preferred_element_type=F32) * rv
    x2u = _dw3x3_half(a2, g2d9, row, col, W, H)

    a3 = jnp.dot(g3p[...], (x1d * x2u).astype(bf),
                 preferred_element_type=F32) * rv
    ff = _dw3x3(a3, g3d9, cl, cr, W)
    out = ff + y
    o_ref[...] = out[:, 4 * W:(T + 4) * W]


def _dw9(w):
    C = w.shape[0]
    return jnp.transpose(w.reshape(C, 9), (1, 0))[:, :, None]


def kernel(x, qkv_w, qkv_dw_w, proj_w, temperature, attn_mix, ln_w, ln_b,
           ff_in_w, g1_pw, g1_dw, g2_pw, g2_dw, g3_pw, g3_dw):
    b, C, H, W = x.shape
    N = H * W
    xf = x.reshape(C, N)
    bf = jnp.bfloat16
    wqkv = qkv_w.reshape(3 * C, C).astype(bf)
    wdw9 = _dw9(qkv_dw_w)
    projw = proj_w.reshape(C, C).astype(bf)
    ffw = ff_in_w.reshape(2 * C, C).astype(bf)
    g1p = g1_pw.reshape(C, C).astype(bf)
    g2p = g2_pw.reshape(C, C).astype(bf)
    g3p = g3_pw.reshape(C, C).astype(bf)
    g1d9 = _dw9(g1_dw)
    g2d9 = _dw9(g2_dw)
    g3d9 = _dw9(g3_dw)
    lnw = ln_w.reshape(C, 1)
    lnb = ln_b.reshape(C, 1)
    temp = temperature.reshape(HEADS, 1)
    mix = attn_mix.reshape(4, 1)

    T1 = 16

    def up_map(i):
        return (0, jnp.maximum(i * T1 - 1, 0))

    def dn_map(i):
        return (0, jnp.minimum(i * T1 + T1, H - 1))

    halo_spec = pl.BlockSpec((C, W), up_map)
    halo_spec_dn = pl.BlockSpec((C, W), dn_map)
    main_spec = pl.BlockSpec((C, T1 * W), lambda i: (0, i))
    full2 = lambda s: pl.BlockSpec(s, lambda i: (0, 0))
    full3 = lambda s: pl.BlockSpec(s, lambda i: (0, 0, 0))

    v, S, nq, nk = pl.pallas_call(
        functools.partial(_k1_body, T=T1, W=W, H=H, C=C),
        grid=(H // T1,),
        in_specs=[halo_spec, main_spec, halo_spec_dn,
                  full2((3 * C, C)), full3((9, 3 * C, 1))],
        out_specs=[main_spec, full2((C, C)), full2((C, 128)),
                   full2((C, 128))],
        out_shape=[jax.ShapeDtypeStruct((C, N), F32),
                   jax.ShapeDtypeStruct((C, C), F32),
                   jax.ShapeDtypeStruct((C, 128), F32),
                   jax.ShapeDtypeStruct((C, 128), F32)],
    )(xf, xf, xf, wqkv, wdw9)

    spa = pl.pallas_call(
        functools.partial(_k2_body, C=C),
        grid=(1,),
        in_specs=[full2((C, C)), full2((C, 128)), full2((C, 128)),
                  full2((HEADS, 1)), full2((4, 1))],
        out_specs=full2((C, C)),
        out_shape=jax.ShapeDtypeStruct((C, C), F32),
    )(S, nq, nk, temp, mix)

    T3 = 16
    nb3 = T3 // 4
    nrb4 = H // 4

    def up3(i):
        return (0, jnp.maximum(i * nb3 - 1, 0))

    def dn3(i):
        return (0, jnp.minimum(i * nb3 + nb3, nrb4 - 1))

    halo3u = pl.BlockSpec((C, 4 * W), up3)
    halo3d = pl.BlockSpec((C, 4 * W), dn3)
    main3 = pl.BlockSpec((C, T3 * W), lambda i: (0, i))

    out = pl.pallas_call(
        functools.partial(_k3_body, T=T3, W=W, H=H, C=C),
        grid=(H // T3,),
        in_specs=[halo3u, main3, halo3d,
                  halo3u, main3, halo3d,
                  full2((C, C)), full2((C, C)), full2((C, 1)), full2((C, 1)),
                  full2((2 * C, C)),
                  full2((C, C)), full3((9, C, 1)),
                  full2((C, C)), full3((9, C, 1)),
                  full2((C, C)), full3((9, C, 1))],
        out_specs=main3,
        out_shape=jax.ShapeDtypeStruct((C, N), F32),
    )(xf, xf, xf, v, v, v, spa, projw, lnw, lnb, ffw,
      g1p, g1d9, g2p, g2d9, g3p, g3d9)

    return out.reshape(b, C, H, W)
